# Initial kernel scaffold; baseline (speedup 1.0000x reference)
import jax
import jax.numpy as jnp
from jax.experimental import pallas as pl


def kernel(x, table):
    raise NotImplementedError("write your pallas kernel here")


# speedup vs baseline: 3.1734x; 3.1734x over previous
import functools

import jax
import jax.numpy as jnp
from jax import lax
from jax.experimental import pallas as pl
from jax.experimental.pallas import tpu as pltpu
from jax.experimental.pallas import tpu_sc as plsc

CHUNK = 128


@functools.partial(jax.jit, static_argnames=("n_workers",))
def _embed_sc(x_flat, table, n_workers):
    b_total = x_flat.shape[0]
    d = table.shape[1]
    per_w = b_total // n_workers
    n_chunks = per_w // CHUNK

    mesh = plsc.VectorSubcoreMesh(core_axis_name="c", subcore_axis_name="s")

    @functools.partial(
        pl.kernel,
        out_type=jax.ShapeDtypeStruct((b_total, d), jnp.float32),
        mesh=mesh,
        scratch_types=[
            pltpu.VMEM((CHUNK,), jnp.int32),
            pltpu.VMEM((CHUNK, d), jnp.float32),
            pltpu.SemaphoreType.DMA,
        ],
        compiler_params=pltpu.CompilerParams(use_tc_tiling_on_sc=False),
    )
    def k(idx_hbm, table_hbm, out_hbm, idx_v, rows_v, sem):
        wid = lax.axis_index("s") * 2 + lax.axis_index("c")
        base = wid * per_w

        def body(c, carry):
            off = base + c * CHUNK
            pltpu.sync_copy(idx_hbm.at[pl.ds(off, CHUNK)], idx_v)
            pltpu.async_copy(table_hbm.at[idx_v], rows_v, sem).wait()
            pltpu.sync_copy(rows_v, out_hbm.at[pl.ds(off, CHUNK)])
            return carry

        lax.fori_loop(0, n_chunks, body, 0)

    return k(x_flat, table)


def kernel(x, table):
    orig_shape = x.shape
    x_flat = x.reshape(-1).astype(jnp.int32)
    out = _embed_sc(x_flat, table, 32)
    return out.reshape(*orig_shape, table.shape[1])


# speedup vs baseline: 4.2504x; 1.3394x over previous
import functools

import jax
import jax.numpy as jnp
from jax import lax
from jax.experimental import pallas as pl
from jax.experimental.pallas import tpu as pltpu
from jax.experimental.pallas import tpu_sc as plsc

CHUNK = 128
NBUF = 4


@functools.partial(jax.jit, static_argnames=("n_workers",))
def _embed_sc(x2d, table, n_workers):
    n_rows_idx = x2d.shape[0]
    b_total = n_rows_idx * CHUNK
    d = table.shape[1]
    per_w = b_total // n_workers
    n_chunks = per_w // CHUNK
    chunks_per_w = n_chunks
    assert n_chunks % NBUF == 0 and n_chunks >= 2 * NBUF

    mesh = plsc.VectorSubcoreMesh(core_axis_name="c", subcore_axis_name="s")

    @functools.partial(
        pl.kernel,
        out_type=jax.ShapeDtypeStruct((b_total, d), jnp.float32),
        mesh=mesh,
        scratch_types=[
            pltpu.VMEM((chunks_per_w, CHUNK), jnp.int32),
            pltpu.VMEM((NBUF, CHUNK, d), jnp.float32),
        ]
        + [pltpu.SemaphoreType.DMA] * (2 * NBUF),
        compiler_params=pltpu.CompilerParams(use_tc_tiling_on_sc=False),
    )
    def k(idx_hbm, table_hbm, out_hbm, idx_v, rows_v, *sems):
        gsem = sems[:NBUF]
        wsem = sems[NBUF:]
        wid = lax.axis_index("s") * 2 + lax.axis_index("c")
        row_base = wid * per_w
        chunk_base = wid * chunks_per_w

        pltpu.sync_copy(idx_hbm.at[pl.ds(chunk_base, chunks_per_w)], idx_v)

        def start_gather(c, b):
            pltpu.make_async_copy(
                table_hbm.at[idx_v.at[c]], rows_v.at[b], gsem[b]
            ).start()

        def wait_gather(b):
            pltpu.make_async_copy(
                table_hbm.at[idx_v.at[0]], rows_v.at[b], gsem[b]
            ).wait()

        def start_write(c, b):
            pltpu.make_async_copy(
                rows_v.at[b], out_hbm.at[pl.ds(row_base + c * CHUNK, CHUNK)],
                wsem[b],
            ).start()

        def wait_write(b):
            pltpu.make_async_copy(
                rows_v.at[b], out_hbm.at[pl.ds(row_base, CHUNK)], wsem[b]
            ).wait()

        for b in range(NBUF):
            start_gather(b, b)

        def block(g, carry):
            for b in range(NBUF):
                wait_gather(b)
                start_write(g + b, b)
            for b in range(NBUF):
                wait_write(b)
                start_gather(g + b + NBUF, b)
            return carry

        lax.fori_loop(0, (n_chunks - NBUF) // NBUF, lambda i, c: block(i * NBUF, c), 0)

        g_last = n_chunks - NBUF
        for b in range(NBUF):
            wait_gather(b)
            start_write(g_last + b, b)
        for b in range(NBUF):
            wait_write(b)

    return k(x2d, table)


def kernel(x, table):
    orig_shape = x.shape
    x2d = x.reshape(-1, CHUNK).astype(jnp.int32)
    out = _embed_sc(x2d, table, 32)
    return out.reshape(*orig_shape, table.shape[1])
